# baseline (speedup 1.0000x reference)
import functools

import jax
import jax.numpy as jnp
from jax import lax
from jax.experimental import pallas as pl
from jax.experimental.pallas import tpu as pltpu
from jax.experimental.pallas import tpu_sc as plsc

_B, _T, _D, _H = 16, 128, 128, 128
_DIM = _H + _D + _H
_CI = 32


def _sc_gather(table, idx):
    n = idx.shape[0]
    d = table.shape[1]
    info = plsc.get_sparse_core_info()
    nc, ns = info.num_cores, info.num_subcores
    per = n // (nc * ns)
    mesh = plsc.VectorSubcoreMesh(core_axis_name="c", subcore_axis_name="s")

    @functools.partial(
        pl.kernel,
        mesh=mesh,
        out_type=jax.ShapeDtypeStruct((n, d), jnp.float32),
        scratch_types=[
            pltpu.VMEM((per,), jnp.int32),
            pltpu.VMEM((per, d), jnp.float32),
            pltpu.SemaphoreType.DMA,
        ],
    )
    def gather_kernel(table_hbm, idx_hbm, out_hbm, idx_v, rows_v, sem):
        wid = lax.axis_index("s") * nc + lax.axis_index("c")
        base = wid * per
        pltpu.sync_copy(idx_hbm.at[pl.ds(base, per)], idx_v)
        pltpu.async_copy(table_hbm.at[idx_v], rows_v, sem).wait()
        pltpu.sync_copy(rows_v, out_hbm.at[pl.ds(base, per)])

    return gather_kernel(table, idx)


def _fused_body(e_ref, wf_ref, uf_ref, bf_ref, wb_ref, ub_ref, bb_ref,
                s_ref, wd_ref, bd_ref, o_ref, xf_ref, xb_ref, x3_ref, sc_ref):
    f32 = jnp.float32
    xf_ref[:] = jnp.dot(e_ref[:], wf_ref[:], preferred_element_type=f32) + bf_ref[:]
    xb_ref[:] = jnp.dot(e_ref[:], wb_ref[:], preferred_element_type=f32) + bb_ref[:]
    x3_ref[:, :, _H:_H + _D] = e_ref[:].reshape(_T, _B, _D)

    def step(t, carry):
        hf, cf, hb, cb = carry
        zf = xf_ref[pl.ds(t * _B, _B), :] + jnp.dot(
            hf, uf_ref[:], preferred_element_type=f32)
        i = jax.nn.sigmoid(zf[:, 0:_H])
        f = jax.nn.sigmoid(zf[:, _H:2 * _H])
        g = jnp.tanh(zf[:, 2 * _H:3 * _H])
        o = jax.nn.sigmoid(zf[:, 3 * _H:4 * _H])
        cf = f * cf + i * g
        hf = o * jnp.tanh(cf)
        x3_ref[t, :, 0:_H] = hf

        zb = xb_ref[pl.ds((_T - 1 - t) * _B, _B), :] + jnp.dot(
            hb, ub_ref[:], preferred_element_type=f32)
        ib = jax.nn.sigmoid(zb[:, 0:_H])
        fb = jax.nn.sigmoid(zb[:, _H:2 * _H])
        gb = jnp.tanh(zb[:, 2 * _H:3 * _H])
        ob = jax.nn.sigmoid(zb[:, 3 * _H:4 * _H])
        cb = fb * cb + ib * gb
        hb = ob * jnp.tanh(cb)
        x3_ref[t, :, _H + _D:_DIM] = hb
        return hf, cf, hb, cb

    def step2(k, carry):
        return step(2 * k + 1, step(2 * k, carry))

    z = jnp.zeros((_B, _H), f32)
    lax.fori_loop(0, _T // 2, step2, (z, z, z, z))

    s_col = s_ref[:]
    s_bf = s_col.astype(jnp.bfloat16)
    wd = wd_ref[:]
    nch = _T // _CI
    for b in range(_B):
        xb = x3_ref[:, b, :]
        xbf = xb.astype(jnp.bfloat16)
        for ic in range(nch):
            xi = xbf[ic * _CI:(ic + 1) * _CI, :]
            for jc in range(ic, nch):
                xj = xbf[jc * _CI:(jc + 1) * _CI, :]
                a = jnp.tanh(xi[:, None, :] + xj[None, :, :])
                blk = jnp.dot(a.reshape(_CI * _CI, _DIM), s_bf,
                              preferred_element_type=f32).reshape(_CI, _CI)
                sc_ref[ic * _CI:(ic + 1) * _CI, jc * _CI:(jc + 1) * _CI] = blk
                if jc > ic:
                    sc_ref[jc * _CI:(jc + 1) * _CI, ic * _CI:(ic + 1) * _CI] = blk.T

        sc = sc_ref[:]
        m = jnp.max(sc, axis=-1, keepdims=True)
        w = jnp.exp(sc - m)
        w = w / jnp.sum(w, axis=-1, keepdims=True)
        attn = jnp.dot(w, xb, preferred_element_type=f32)

        xmean = jnp.mean(xb, axis=0, keepdims=True)
        xmax = jnp.max(xb, axis=0, keepdims=True)
        amean = jnp.mean(attn, axis=0, keepdims=True)
        amax = jnp.max(attn, axis=0, keepdims=True)
        val = jnp.sum(xmean * wd[0:1] + xmax * wd[1:2]
                      + amean * wd[2:3] + amax * wd[3:4]) + bd_ref[0, 0]
        o_ref[b:b + 1, 0:1] = jax.nn.sigmoid(val).reshape(1, 1)


def _fused_call(e_tb, wf, uf, bf, wb, ub, bb, scale, wd, bd):
    return pl.pallas_call(
        _fused_body,
        out_shape=jax.ShapeDtypeStruct((_B, 1), jnp.float32),
        scratch_shapes=[
            pltpu.VMEM((_T * _B, 4 * _H), jnp.float32),
            pltpu.VMEM((_T * _B, 4 * _H), jnp.float32),
            pltpu.VMEM((_T, _B, _DIM), jnp.float32),
            pltpu.VMEM((_T, _T), jnp.float32),
        ],
    )(e_tb, wf, uf, bf, wb, ub, bb, scale, wd, bd)


def kernel(tokens, emb, Wf, Uf, bf, Wb, Ub, bb, attn_scale, Wd, bd):
    idx_tb = tokens.T.reshape(-1).astype(jnp.int32)
    e_tb = _sc_gather(emb, idx_tb)
    return _fused_call(e_tb, Wf, Uf, bf.reshape(1, -1), Wb, Ub,
                       bb.reshape(1, -1), attn_scale.reshape(-1, 1),
                       Wd.reshape(4, _DIM), bd.reshape(1, 1))


# speedup vs baseline: 1.0268x; 1.0268x over previous
import functools

import jax
import jax.numpy as jnp
from jax import lax
from jax.experimental import pallas as pl
from jax.experimental.pallas import tpu as pltpu
from jax.experimental.pallas import tpu_sc as plsc

_B, _T, _D, _H = 16, 128, 128, 128
_DIM = _H + _D + _H
_CI = 32


def _sc_gather(table, idx):
    n = idx.shape[0]
    d = table.shape[1]
    info = plsc.get_sparse_core_info()
    nc, ns = info.num_cores, info.num_subcores
    per = n // (nc * ns)
    mesh = plsc.VectorSubcoreMesh(core_axis_name="c", subcore_axis_name="s")

    @functools.partial(
        pl.kernel,
        mesh=mesh,
        out_type=jax.ShapeDtypeStruct((n, d), jnp.float32),
        scratch_types=[
            pltpu.VMEM((per,), jnp.int32),
            pltpu.VMEM((per, d), jnp.float32),
            pltpu.SemaphoreType.DMA,
        ],
    )
    def gather_kernel(table_hbm, idx_hbm, out_hbm, idx_v, rows_v, sem):
        wid = lax.axis_index("s") * nc + lax.axis_index("c")
        base = wid * per
        pltpu.sync_copy(idx_hbm.at[pl.ds(base, per)], idx_v)
        pltpu.async_copy(table_hbm.at[idx_v], rows_v, sem).wait()
        pltpu.sync_copy(rows_v, out_hbm.at[pl.ds(base, per)])

    return gather_kernel(table, idx)


def _fused_body(e_ref, wf_ref, uf_ref, bf_ref, wb_ref, ub_ref, bb_ref,
                s_ref, wd_ref, bd_ref, o_ref, xf_ref, xb_ref, x3_ref, sc_ref):
    f32 = jnp.float32
    xf_ref[:] = jnp.dot(e_ref[:], wf_ref[:], preferred_element_type=f32) + bf_ref[:]
    xb_ref[:] = jnp.dot(e_ref[:], wb_ref[:], preferred_element_type=f32) + bb_ref[:]
    x3_ref[:, :, _H:_H + _D] = e_ref[:].reshape(_T, _B, _D)

    def step(t, carry):
        hf, cf, hb, cb = carry
        zf = xf_ref[pl.ds(t * _B, _B), :] + jnp.dot(
            hf, uf_ref[:], preferred_element_type=f32)
        i = jax.nn.sigmoid(zf[:, 0:_H])
        f = jax.nn.sigmoid(zf[:, _H:2 * _H])
        g = jnp.tanh(zf[:, 2 * _H:3 * _H])
        o = jax.nn.sigmoid(zf[:, 3 * _H:4 * _H])
        cf = f * cf + i * g
        hf = o * jnp.tanh(cf)
        x3_ref[t, :, 0:_H] = hf

        zb = xb_ref[pl.ds((_T - 1 - t) * _B, _B), :] + jnp.dot(
            hb, ub_ref[:], preferred_element_type=f32)
        ib = jax.nn.sigmoid(zb[:, 0:_H])
        fb = jax.nn.sigmoid(zb[:, _H:2 * _H])
        gb = jnp.tanh(zb[:, 2 * _H:3 * _H])
        ob = jax.nn.sigmoid(zb[:, 3 * _H:4 * _H])
        cb = fb * cb + ib * gb
        hb = ob * jnp.tanh(cb)
        x3_ref[t, :, _H + _D:_DIM] = hb
        return hf, cf, hb, cb

    def step4(k, carry):
        for j in range(4):
            carry = step(4 * k + j, carry)
        return carry

    z = jnp.zeros((_B, _H), f32)
    lax.fori_loop(0, _T // 4, step4, (z, z, z, z))

    s_col = s_ref[:]
    wd = wd_ref[:]
    nch = _T // _CI
    for b in range(_B):
        xb = x3_ref[:, b, :]
        for ic in range(nch):
            xi = xb[ic * _CI:(ic + 1) * _CI, :]
            for jc in range(ic, nch):
                xj = xb[jc * _CI:(jc + 1) * _CI, :]
                a = jnp.tanh(xi[:, None, :] + xj[None, :, :])
                blk = jnp.dot(a.reshape(_CI * _CI, _DIM), s_col,
                              preferred_element_type=f32).reshape(_CI, _CI)
                sc_ref[ic * _CI:(ic + 1) * _CI, jc * _CI:(jc + 1) * _CI] = blk
                if jc > ic:
                    sc_ref[jc * _CI:(jc + 1) * _CI, ic * _CI:(ic + 1) * _CI] = blk.T

        sc = sc_ref[:]
        m = jnp.max(sc, axis=-1, keepdims=True)
        w = jnp.exp(sc - m)
        w = w / jnp.sum(w, axis=-1, keepdims=True)
        attn = jnp.dot(w, xb, preferred_element_type=f32)

        xmean = jnp.mean(xb, axis=0, keepdims=True)
        xmax = jnp.max(xb, axis=0, keepdims=True)
        amean = jnp.mean(attn, axis=0, keepdims=True)
        amax = jnp.max(attn, axis=0, keepdims=True)
        val = jnp.sum(xmean * wd[0:1] + xmax * wd[1:2]
                      + amean * wd[2:3] + amax * wd[3:4]) + bd_ref[0, 0]
        o_ref[b:b + 1, 0:1] = jax.nn.sigmoid(val).reshape(1, 1)


def _fused_call(e_tb, wf, uf, bf, wb, ub, bb, scale, wd, bd):
    return pl.pallas_call(
        _fused_body,
        out_shape=jax.ShapeDtypeStruct((_B, 1), jnp.float32),
        scratch_shapes=[
            pltpu.VMEM((_T * _B, 4 * _H), jnp.float32),
            pltpu.VMEM((_T * _B, 4 * _H), jnp.float32),
            pltpu.VMEM((_T, _B, _DIM), jnp.float32),
            pltpu.VMEM((_T, _T), jnp.float32),
        ],
    )(e_tb, wf, uf, bf, wb, ub, bb, scale, wd, bd)


def kernel(tokens, emb, Wf, Uf, bf, Wb, Ub, bb, attn_scale, Wd, bd):
    idx_tb = tokens.T.reshape(-1).astype(jnp.int32)
    e_tb = _sc_gather(emb, idx_tb)
    return _fused_call(e_tb, Wf, Uf, bf.reshape(1, -1), Wb, Ub,
                       bb.reshape(1, -1), attn_scale.reshape(-1, 1),
                       Wd.reshape(4, _DIM), bd.reshape(1, 1))
